# baseline (speedup 1.0000x reference)
import jax
import jax.numpy as jnp
from jax import lax
from jax.experimental import pallas as pl
from jax.experimental.pallas import tpu as pltpu
from jax.experimental.pallas import tpu_sc as plsc

NIN = 50000
NOUT = 100000
HID = 128

L = 16
NC = 2
NS = 16
NW = NC * NS

G_TOTAL = NOUT // L
RANKS = 16
RG_BASE = 195
RG_EXTRA = 10
SC_SPAN = 8 * (RG_BASE + 1) * L
HALF_G = 98
HALF_ROWS = HALF_G * L
LIST_PAD = HALF_ROWS + L

N_VECS = NIN // L
SV_BASE = 195
SV_EXTRA = 5

UZ_START = RANKS * RG_BASE + RG_EXTRA
UZ_BASE = 97
UZ_EXTRA = 16

NB = 8
SEG = HALF_ROWS
SPM_RAW = NS * SC_SPAN
SPM_TOTAL = SPM_RAW + SC_SPAN


def _rank_base_g(r):
  return r * RG_BASE + jnp.minimum(r, RG_EXTRA)


def _body(feat_hbm, src_hbm, dst_hbm, out_hbm,
          src_v, dst_sl, win_big, win_v, vidx_v, gidx_v, zidx_v,
          zrow_v, gbuf_v, spm,
          usem, srcsem, psem, zsem, gsem, ssem):
  cid = lax.axis_index("c")
  sid = lax.axis_index("s")

  iota = lax.iota(jnp.int32, L)
  zeros_f = jnp.zeros((L,), jnp.float32)
  neg1 = jnp.full((L,), -1, jnp.int32)
  zeros_i = jnp.zeros((L,), jnp.int32)

  for r in range(L):
    for c in range(HID // L):
      zrow_v[r, pl.ds(c * L, L)] = zeros_f

  uk = cid * NS + sid
  ubase_g = UZ_START + uk * UZ_BASE + jnp.minimum(uk, UZ_EXTRA)
  un_g = UZ_BASE + jnp.where(uk < UZ_EXTRA, 1, 0)

  def uz_body(g, carry):
    rb = (ubase_g + g) * L
    pltpu.async_copy(zrow_v, out_hbm.at[pl.ds(rb, L), :], usem)
    return carry
  lax.fori_loop(0, un_g, uz_body, 0)

  src_copy = pltpu.async_copy(src_hbm, src_v, srcsem)

  with jax.named_scope("p_init"):
    def init_win(v, carry):
      win_big[pl.ds(v * L, L)] = neg1
      return carry
    lax.fori_loop(0, SC_SPAN // L, init_win, 0)

  scopeh = jax.named_scope("p_scan")
  scopeh.__enter__()
  vstart = sid * SV_BASE + jnp.minimum(sid, SV_EXTRA)
  vcount = SV_BASE + jnp.where(sid < SV_EXTRA, 1, 0)
  coff = jnp.minimum(vstart * L, NIN - SV_BASE * L - L)
  delta = vstart * L - coff
  pltpu.sync_copy(dst_hbm.at[pl.ds(coff, (SV_BASE + 1) * L)], dst_sl)

  sc_base = cid * SC_SPAN

  def scan_body(v, carry):
    d = dst_sl[pl.ds(delta + v * L, L)]
    inr = (d >= sc_base) & (d < sc_base + SC_SPAN)
    _, last = plsc.scan_count(d, mask=inr)
    m = last & inr
    loc = jnp.where(m, d - sc_base, 0)
    ivec = (vstart + v) * L + iota
    plsc.store_scatter(win_big, [loc], ivec, mask=m)
    return carry
  lax.fori_loop(0, vcount, scan_body, 0)

  scopeh.__exit__(None, None, None)
  with jax.named_scope("p_publish"):
    pltpu.sync_copy(win_big, spm.at[pl.ds(sid * SC_SPAN, SC_SPAN)])
    plsc.subcore_barrier()

  scopem = jax.named_scope("p_merge")
  scopem.__enter__()
  for s in range(NS):
    pltpu.async_copy(
        spm.at[pl.ds(s * SC_SPAN + sid * SEG, SEG)],
        win_big.at[pl.ds(s * SEG, SEG)], psem)
  for s in range(NS):
    pltpu.make_async_copy(
        spm.at[pl.ds(sid * SEG, SEG)],
        win_big.at[pl.ds(0, SEG)], psem).wait()

  def merge_vec(v, carry):
    acc = win_big[pl.ds(v * L, L)]
    for s in range(1, NS):
      ws = win_big[pl.ds(s * SEG + v * L, L)]
      acc = jnp.where(ws >= 0, ws, acc)
    dst_sl[pl.ds(v * L, L)] = acc
    return carry
  lax.fori_loop(0, SEG // L, merge_vec, 0)

  pltpu.sync_copy(dst_sl.at[pl.ds(0, SEG)],
                  spm.at[pl.ds(SPM_RAW + sid * SEG, SEG)])
  plsc.subcore_barrier()
  scopem.__exit__(None, None, None)

  scopec = jax.named_scope("p_compact")
  scopec.__enter__()
  rank = cid * 8 + jnp.where(sid < 8, sid, sid - 8)
  rbase_g = _rank_base_g(rank)
  rn_g = RG_BASE + jnp.where(rank < RG_EXTRA, 1, 0)
  h = rn_g // 2
  is_owner = sid < 8
  my_g0 = jnp.where(is_owner, 0, h)
  ng_me = jnp.where(is_owner, h, rn_g - h)
  my_base_row = (rbase_g + my_g0) * L
  span_off = rbase_g * L - sc_base + my_g0 * L

  pltpu.sync_copy(spm.at[pl.ds(SPM_RAW + span_off, HALF_ROWS)], win_v)
  src_copy.wait()

  def compact_body(g, carry):
    nvo, nzo = carry
    w16 = win_v[pl.ds(g * L, L)]
    valid = w16 >= 0
    oid = my_base_row + g * L + iota
    cs_v = plsc.cumsum(jnp.where(valid, 1, 0))
    cs_z = plsc.cumsum(jnp.where(valid, 0, 1))
    pv = jnp.max(cs_v)
    plsc.store_scatter(vidx_v, [nvo + cs_v - 1], oid, mask=valid)
    gi = plsc.load_gather(src_v, [jnp.where(valid, w16, 0)])
    plsc.store_scatter(gidx_v, [nvo + cs_v - 1], gi, mask=valid)
    plsc.store_scatter(zidx_v, [nzo + cs_z - 1], oid,
                       mask=jnp.logical_not(valid))
    return nvo + pv, nzo + (L - pv)
  nvo, nzo = lax.fori_loop(0, ng_me, compact_body, (0, 0))

  plsc.store_scatter(vidx_v, [nvo + iota],
                     plsc.load_gather(vidx_v, [zeros_i]))
  plsc.store_scatter(gidx_v, [nvo + iota],
                     plsc.load_gather(gidx_v, [zeros_i]))
  plsc.store_scatter(zidx_v, [nzo + iota],
                     plsc.load_gather(zidx_v, [zeros_i]))

  scopec.__exit__(None, None, None)
  scopee = jax.named_scope("p_emit")
  scopee.__enter__()
  nzb = (nzo + L - 1) // L

  def zfire(k, carry):
    zI = zidx_v[pl.ds(k * L, L)]
    pltpu.async_copy(zrow_v, out_hbm.at[zI], zsem)
    return carry
  lax.fori_loop(0, nzb, zfire, 0)

  nvb = (nvo + L - 1) // L
  full = nvb // NB

  def blk_body(blk, carry):
    for b in range(NB):
      @pl.when(blk > 0)
      def _():
        pltpu.make_async_copy(
            gbuf_v.at[b], out_hbm.at[pl.ds(0, L), :], ssem[b]).wait()
      gI = gidx_v[pl.ds((blk * NB + b) * L, L)]
      pltpu.async_copy(feat_hbm.at[gI], gbuf_v.at[b], gsem[b])
    for b in range(NB):
      pltpu.make_async_copy(
          feat_hbm.at[pl.ds(0, L), :], gbuf_v.at[b], gsem[b]).wait()
      oI = vidx_v[pl.ds((blk * NB + b) * L, L)]
      pltpu.async_copy(gbuf_v.at[b], out_hbm.at[oI], ssem[b])
    return carry
  lax.fori_loop(0, full, blk_body, 0)

  for b in range(NB):
    @pl.when(full > 0)
    def _():
      pltpu.make_async_copy(
          gbuf_v.at[b], out_hbm.at[pl.ds(0, L), :], ssem[b]).wait()

  def tail_body(k, carry):
    gI = gidx_v[pl.ds(k * L, L)]
    pltpu.async_copy(feat_hbm.at[gI], gbuf_v.at[0], gsem[0]).wait()
    oI = vidx_v[pl.ds(k * L, L)]
    pltpu.async_copy(gbuf_v.at[0], out_hbm.at[oI], ssem[0]).wait()
    return carry
  lax.fori_loop(full * NB, nvb, tail_body, 0)

  scopee.__exit__(None, None, None)
  def zdrain(k, carry):
    pltpu.make_async_copy(
        out_hbm.at[pl.ds(0, L), :], gbuf_v.at[0], zsem).wait()
    return carry
  lax.fori_loop(0, nzb, zdrain, 0)

  def udrain(g, carry):
    pltpu.make_async_copy(
        out_hbm.at[pl.ds(0, L), :], gbuf_v.at[0], usem).wait()
    return carry
  lax.fori_loop(0, un_g, udrain, 0)


@jax.jit
def kernel(feat, idxs):
  src = idxs[0]
  dst = idxs[1]
  mesh = plsc.VectorSubcoreMesh(core_axis_name="c", subcore_axis_name="s")
  run = pl.kernel(
      _body,
      out_type=jax.ShapeDtypeStruct((NOUT, HID), jnp.float32),
      mesh=mesh,
      compiler_params=pltpu.CompilerParams(needs_layout_passes=False),
      scratch_types=[
          pltpu.VMEM((NIN,), jnp.int32),
          pltpu.VMEM(((SV_BASE + 1) * L,), jnp.int32),
          pltpu.VMEM((NS * SEG,), jnp.int32),
          pltpu.VMEM((HALF_ROWS,), jnp.int32),
          pltpu.VMEM((LIST_PAD,), jnp.int32),
          pltpu.VMEM((LIST_PAD,), jnp.int32),
          pltpu.VMEM((LIST_PAD,), jnp.int32),
          pltpu.VMEM((L, HID), jnp.float32),
          pltpu.VMEM((NB, L, HID), jnp.float32),
          pltpu.VMEM_SHARED((SPM_TOTAL,), jnp.int32),
          pltpu.SemaphoreType.DMA,
          pltpu.SemaphoreType.DMA,
          pltpu.SemaphoreType.DMA,
          pltpu.SemaphoreType.DMA,
          [pltpu.SemaphoreType.DMA] * NB,
          [pltpu.SemaphoreType.DMA] * NB,
      ],
  )
  return run(feat, src, dst)


# speedup vs baseline: 1.2456x; 1.2456x over previous
import jax
import jax.numpy as jnp
from jax import lax
from jax.experimental import pallas as pl
from jax.experimental.pallas import tpu as pltpu
from jax.experimental.pallas import tpu_sc as plsc

NIN = 50000
NOUT = 100000
HID = 128

L = 16
NC = 2
NS = 16
NW = NC * NS

G_TOTAL = NOUT // L
RANKS = 16
RG_BASE = 195
RG_EXTRA = 10
SC_SPAN = 8 * (RG_BASE + 1) * L
HALF_G = 98
HALF_ROWS = HALF_G * L
BLK = 64
EG = 128
LIST_PAD = HALF_ROWS + BLK
GIDX_PAD = HALF_ROWS + EG

N_VECS = NIN // L
SV_BASE = 195
SV_EXTRA = 5

UZ_START = RANKS * RG_BASE + RG_EXTRA
UZ_BASE = 97
UZ_EXTRA = 16

NB = 4
SEG = HALF_ROWS
SPM_RAW = NS * SC_SPAN
SPM_TOTAL = SPM_RAW + SC_SPAN


def _rank_base_g(r):
  return r * RG_BASE + jnp.minimum(r, RG_EXTRA)


def _body(feat_hbm, src_hbm, dst_hbm, out_hbm,
          dst_sl, win_big, win_v, vidx_v, gidx_v, gval_v, zidx_v,
          zrow_v, gbuf_v, spm,
          usem, srcsem, psem, zsem, gsem, ssem):
  cid = lax.axis_index("c")
  sid = lax.axis_index("s")

  iota = lax.iota(jnp.int32, L)
  zeros_f = jnp.zeros((L,), jnp.float32)
  neg1 = jnp.full((L,), -1, jnp.int32)
  zeros_i = jnp.zeros((L,), jnp.int32)

  def zb_body(r, carry):
    for c in range(HID // L):
      zrow_v[r, pl.ds(c * L, L)] = zeros_f
    return carry
  lax.fori_loop(0, BLK, zb_body, 0)

  uk = cid * NS + sid
  ubase_g = UZ_START + uk * UZ_BASE + jnp.minimum(uk, UZ_EXTRA)
  un_g = UZ_BASE + jnp.where(uk < UZ_EXTRA, 1, 0)

  un_rows = un_g * L
  ub_big = un_rows // BLK
  ub_rem = (un_rows - ub_big * BLK) // L

  def uz_body(k, carry):
    rb = ubase_g * L + k * BLK
    pltpu.async_copy(zrow_v, out_hbm.at[pl.ds(rb, BLK), :], usem)
    return carry
  lax.fori_loop(0, ub_big, uz_body, 0)

  def uz_rem(k, carry):
    rb = ubase_g * L + ub_big * BLK + k * L
    pltpu.async_copy(zrow_v.at[pl.ds(0, L), :],
                     out_hbm.at[pl.ds(rb, L), :], usem)
    return carry
  lax.fori_loop(0, ub_rem, uz_rem, 0)

  def init_gidx(v, carry):
    gidx_v[pl.ds(v * L, L)] = zeros_i
    return carry
  lax.fori_loop(0, GIDX_PAD // L, init_gidx, 0)

  def init_win(v, carry):
    for u in range(8):
      win_big[pl.ds((v * 8 + u) * L, L)] = neg1
    return carry
  lax.fori_loop(0, SC_SPAN // (8 * L), init_win, 0)

  vstart = sid * SV_BASE + jnp.minimum(sid, SV_EXTRA)
  vcount = SV_BASE + jnp.where(sid < SV_EXTRA, 1, 0)
  coff = jnp.minimum(vstart * L, NIN - SV_BASE * L - L)
  delta = vstart * L - coff
  pltpu.sync_copy(dst_hbm.at[pl.ds(coff, (SV_BASE + 1) * L)], dst_sl)

  sc_base = cid * SC_SPAN

  def scan_one(v):
    d = dst_sl[pl.ds(delta + v * L, L)]
    inr = (d >= sc_base) & (d < sc_base + SC_SPAN)
    _, last = plsc.scan_count(d, mask=inr)
    m = last & inr
    loc = jnp.where(m, d - sc_base, 0)
    ivec = (vstart + v) * L + iota
    plsc.store_scatter(win_big, [loc], ivec, mask=m)

  def scan_body(v, carry):
    scan_one(2 * v)
    scan_one(2 * v + 1)
    return carry
  lax.fori_loop(0, vcount // 2, scan_body, 0)

  @pl.when(vcount % 2 == 1)
  def _():
    scan_one(vcount - 1)

  pltpu.sync_copy(win_big, spm.at[pl.ds(sid * SC_SPAN, SC_SPAN)])
  plsc.subcore_barrier()

  for s in range(NS):
    pltpu.async_copy(
        spm.at[pl.ds(s * SC_SPAN + sid * SEG, SEG)],
        win_big.at[pl.ds(s * SEG, SEG)], psem)
  for s in range(NS):
    pltpu.make_async_copy(
        spm.at[pl.ds(sid * SEG, SEG)],
        win_big.at[pl.ds(0, SEG)], psem).wait()

  def merge_vec(v, carry):
    acc = win_big[pl.ds(v * L, L)]
    for s in range(1, NS):
      ws = win_big[pl.ds(s * SEG + v * L, L)]
      acc = jnp.where(ws >= 0, ws, acc)
    dst_sl[pl.ds(v * L, L)] = acc
    return carry
  lax.fori_loop(0, SEG // L, merge_vec, 0)

  pltpu.sync_copy(dst_sl.at[pl.ds(0, SEG)],
                  spm.at[pl.ds(SPM_RAW + sid * SEG, SEG)])
  plsc.subcore_barrier()

  rank = cid * 8 + jnp.where(sid < 8, sid, sid - 8)
  rbase_g = _rank_base_g(rank)
  rn_g = RG_BASE + jnp.where(rank < RG_EXTRA, 1, 0)
  h = rn_g // 2
  is_owner = sid < 8
  my_g0 = jnp.where(is_owner, 0, h)
  ng_me = jnp.where(is_owner, h, rn_g - h)
  my_base_row = (rbase_g + my_g0) * L
  span_off = rbase_g * L - sc_base + my_g0 * L

  pltpu.sync_copy(spm.at[pl.ds(SPM_RAW + span_off, HALF_ROWS)], win_v)

  def compact_body(g, carry):
    nvo, nzo = carry
    w16 = win_v[pl.ds(g * L, L)]
    valid = w16 >= 0
    oid = my_base_row + g * L + iota
    cs_v = plsc.cumsum(jnp.where(valid, 1, 0))
    cs_z = (iota + 1) - cs_v
    pv = jnp.max(cs_v)
    plsc.store_scatter(vidx_v, [nvo + cs_v - 1], oid, mask=valid)
    plsc.store_scatter(gidx_v, [nvo + cs_v - 1], w16, mask=valid)
    plsc.store_scatter(zidx_v, [nzo + cs_z - 1], oid,
                       mask=jnp.logical_not(valid))
    return nvo + pv, nzo + (L - pv)
  nvo, nzo = lax.fori_loop(0, ng_me, compact_body, (0, 0))

  for j in range(BLK // L):
    plsc.store_scatter(vidx_v, [nvo + j * L + iota],
                       plsc.load_gather(vidx_v, [zeros_i]))
    plsc.store_scatter(gidx_v, [nvo + j * L + iota],
                       plsc.load_gather(gidx_v, [zeros_i]))
    plsc.store_scatter(zidx_v, [nzo + j * L + iota],
                       plsc.load_gather(zidx_v, [zeros_i]))

  nvg = (nvo + BLK - 1) // BLK
  mg = (nvg * BLK + EG - 1) // EG

  def eg_fire(m, carry):
    pltpu.async_copy(src_hbm.at[gidx_v.at[pl.ds(m * EG, EG)]],
                     gval_v.at[pl.ds(m * EG, EG)], srcsem)
    return carry
  lax.fori_loop(0, mg, eg_fire, 0)

  def eg_drain(m, carry):
    pltpu.make_async_copy(src_hbm.at[pl.ds(0, EG)],
                          gval_v.at[pl.ds(0, EG)], srcsem).wait()
    return carry
  lax.fori_loop(0, mg, eg_drain, 0)

  nzb = (nzo + L - 1) // L

  def zfire(k, carry):
    zI = zidx_v[pl.ds(k * L, L)]
    pltpu.async_copy(zrow_v.at[pl.ds(0, L), :], out_hbm.at[zI], zsem)
    return carry
  lax.fori_loop(0, nzb, zfire, 0)

  full = nvg // NB

  def blk_body(blk, carry):
    for b in range(NB):
      @pl.when(blk > 0)
      def _():
        for j in range(BLK // L):
          pltpu.make_async_copy(
              gbuf_v.at[b, pl.ds(0, L), :], out_hbm.at[pl.ds(0, L), :],
              ssem[b]).wait()
      k = blk * NB + b
      pltpu.async_copy(
          feat_hbm.at[gval_v.at[pl.ds(k * BLK, BLK)]], gbuf_v.at[b],
          gsem[b])
    for b in range(NB):
      pltpu.make_async_copy(
          feat_hbm.at[pl.ds(0, BLK), :], gbuf_v.at[b], gsem[b]).wait()
      k = blk * NB + b
      for j in range(BLK // L):
        oI = vidx_v[pl.ds(k * BLK + j * L, L)]
        pltpu.async_copy(gbuf_v.at[b, pl.ds(j * L, L), :],
                         out_hbm.at[oI], ssem[b])
    return carry
  lax.fori_loop(0, full, blk_body, 0)

  for b in range(NB):
    @pl.when(full > 0)
    def _():
      for j in range(BLK // L):
        pltpu.make_async_copy(
            gbuf_v.at[b, pl.ds(0, L), :], out_hbm.at[pl.ds(0, L), :],
            ssem[b]).wait()

  def tail_body(k, carry):
    pltpu.async_copy(
        feat_hbm.at[gval_v.at[pl.ds(k * BLK, BLK)]], gbuf_v.at[0],
        gsem[0]).wait()
    for j in range(BLK // L):
      oI = vidx_v[pl.ds(k * BLK + j * L, L)]
      pltpu.async_copy(gbuf_v.at[0, pl.ds(j * L, L), :],
                       out_hbm.at[oI], ssem[0])
    for j in range(BLK // L):
      pltpu.make_async_copy(
          gbuf_v.at[0, pl.ds(0, L), :], out_hbm.at[pl.ds(0, L), :],
          ssem[0]).wait()
    return carry
  lax.fori_loop(full * NB, nvg, tail_body, 0)

  def zdrain(k, carry):
    pltpu.make_async_copy(
        out_hbm.at[pl.ds(0, L), :], gbuf_v.at[0, pl.ds(0, L), :],
        zsem).wait()
    return carry
  lax.fori_loop(0, nzb, zdrain, 0)

  def udrain(k, carry):
    pltpu.make_async_copy(
        out_hbm.at[pl.ds(0, BLK), :], gbuf_v.at[0], usem).wait()
    return carry
  lax.fori_loop(0, ub_big, udrain, 0)

  def udrain_rem(k, carry):
    pltpu.make_async_copy(
        out_hbm.at[pl.ds(0, L), :], gbuf_v.at[0, pl.ds(0, L), :],
        usem).wait()
    return carry
  lax.fori_loop(0, ub_rem, udrain_rem, 0)


@jax.jit
def kernel(feat, idxs):
  src = idxs[0]
  dst = idxs[1]
  mesh = plsc.VectorSubcoreMesh(core_axis_name="c", subcore_axis_name="s")
  run = pl.kernel(
      _body,
      out_type=jax.ShapeDtypeStruct((NOUT, HID), jnp.float32),
      mesh=mesh,
      compiler_params=pltpu.CompilerParams(needs_layout_passes=False),
      scratch_types=[
          pltpu.VMEM(((SV_BASE + 1) * L,), jnp.int32),
          pltpu.VMEM((NS * SEG,), jnp.int32),
          pltpu.VMEM((HALF_ROWS,), jnp.int32),
          pltpu.VMEM((LIST_PAD,), jnp.int32),
          pltpu.VMEM((GIDX_PAD,), jnp.int32),
          pltpu.VMEM((GIDX_PAD,), jnp.int32),
          pltpu.VMEM((LIST_PAD,), jnp.int32),
          pltpu.VMEM((BLK, HID), jnp.float32),
          pltpu.VMEM((NB, BLK, HID), jnp.float32),
          pltpu.VMEM_SHARED((SPM_TOTAL,), jnp.int32),
          pltpu.SemaphoreType.DMA,
          pltpu.SemaphoreType.DMA,
          pltpu.SemaphoreType.DMA,
          pltpu.SemaphoreType.DMA,
          [pltpu.SemaphoreType.DMA] * NB,
          [pltpu.SemaphoreType.DMA] * NB,
      ],
  )
  return run(feat, src, dst)
